# baseline (speedup 1.0000x reference)
import functools

import jax
import jax.numpy as jnp
from jax import lax
from jax.experimental import pallas as pl
from jax.experimental.pallas import tpu as pltpu
from jax.experimental.pallas import tpu_sc as plsc

_NC = 2
_NS = 16
_NW = _NC * _NS


@functools.lru_cache(maxsize=None)
def _build(B, S, D):
    assert B % (2 * _NW) == 0
    b_rows = B // _NW
    mesh = plsc.VectorSubcoreMesh(
        core_axis_name="c", subcore_axis_name="s",
        num_cores=_NC, num_subcores=_NS)

    @functools.partial(
        pl.kernel,
        out_type=jax.ShapeDtypeStruct((B, S, D), jnp.float32),
        mesh=mesh,
        scratch_types=[
            pltpu.VMEM((b_rows, S), jnp.int32),
            pltpu.VMEM((S, D), jnp.float32),
            pltpu.VMEM((S, D), jnp.float32),
            pltpu.SemaphoreType.DMA,
            pltpu.SemaphoreType.DMA,
            pltpu.SemaphoreType.DMA,
            pltpu.SemaphoreType.DMA,
        ],
        compiler_params=pltpu.CompilerParams(use_tc_tiling_on_sc=False),
    )
    def gather_kernel(idx_hbm, table_hbm, out_hbm,
                      idx_v, rows0, rows1, sg0, sg1, so0, so1):
        wid = lax.axis_index("s") * _NC + lax.axis_index("c")
        base = wid * b_rows
        bufs = ((rows0, sg0, so0), (rows1, sg1, so1))

        pltpu.sync_copy(idx_hbm.at[pl.ds(pl.multiple_of(base, b_rows),
                                         b_rows)], idx_v)

        def gather(r, b):
            rows, sg, _ = bufs[b]
            return pltpu.make_async_copy(
                table_hbm.at[idx_v.at[r]], rows, sg)

        def writeback(r, b):
            rows, _, so = bufs[b]
            return pltpu.make_async_copy(rows, out_hbm.at[base + r], so)

        def outer(k, _):
            for j in range(2):
                r = 2 * k + j
                b = j
                bp = 1 - j

                @pl.when(k >= 1)
                def _():
                    writeback(r - 2, b).wait()
                gather(r, b).start()

                if j == 0:
                    @pl.when(k >= 1)
                    def _():
                        gather(r - 1, bp).wait()
                        writeback(r - 1, bp).start()
                else:
                    gather(r - 1, bp).wait()
                    writeback(r - 1, bp).start()
            return ()

        lax.fori_loop(0, b_rows // 2, outer, (), unroll=False)

        gather(b_rows - 1, 1).wait()
        writeback(b_rows - 1, 1).start()
        writeback(b_rows - 2, 0).wait()
        writeback(b_rows - 1, 1).wait()

    return gather_kernel


@jax.jit
def kernel(inputs, embedding):
    B, S = inputs.shape
    _, D = embedding.shape
    return _build(B, S, D)(inputs, embedding)


# speedup vs baseline: 1.0008x; 1.0008x over previous
import functools

import jax
import jax.numpy as jnp
from jax import lax
from jax.experimental import pallas as pl
from jax.experimental.pallas import tpu as pltpu
from jax.experimental.pallas import tpu_sc as plsc

_NC = 2
_NS = 16
_NW = _NC * _NS
_D = 64
_C = 800


@functools.lru_cache(maxsize=None)
def _build(B):
    assert B % (_NW * 2 * _C) == 0
    b_per_w = B // _NW
    n_chunks = b_per_w // _C
    mesh = plsc.VectorSubcoreMesh(
        core_axis_name="c", subcore_axis_name="s",
        num_cores=_NC, num_subcores=_NS)

    @functools.partial(
        pl.kernel,
        out_type=jax.ShapeDtypeStruct((B, _D), jnp.float32),
        mesh=mesh,
        scratch_types=[
            pltpu.VMEM((b_per_w,), jnp.int32),
            pltpu.VMEM((_C, _D), jnp.float32),
            pltpu.VMEM((_C, _D), jnp.float32),
            pltpu.SemaphoreType.DMA,
            pltpu.SemaphoreType.DMA,
            pltpu.SemaphoreType.DMA,
            pltpu.SemaphoreType.DMA,
        ],
        compiler_params=pltpu.CompilerParams(use_tc_tiling_on_sc=False),
    )
    def gather_kernel(idx_hbm, table_hbm, out_hbm,
                      idx_v, rows0, rows1, sg0, sg1, so0, so1):
        wid = lax.axis_index("s") * _NC + lax.axis_index("c")
        base = wid * b_per_w
        bufs = ((rows0, sg0, so0), (rows1, sg1, so1))

        pltpu.sync_copy(idx_hbm.at[pl.ds(pl.multiple_of(base, _C), b_per_w)],
                        idx_v)

        def gather(g, b):
            rows, sg, _ = bufs[b]
            loc = pl.multiple_of(g * _C, _C)
            return pltpu.make_async_copy(
                table_hbm.at[idx_v.at[pl.ds(loc, _C)]], rows, sg)

        def writeback(g, b):
            rows, _, so = bufs[b]
            off = pl.multiple_of(base + g * _C, _C)
            return pltpu.make_async_copy(rows, out_hbm.at[pl.ds(off, _C)], so)

        def outer(k, _):
            for j in range(2):
                g = 2 * k + j
                b = j
                bp = 1 - j

                @pl.when(k >= 1)
                def _():
                    writeback(g - 2, b).wait()
                gather(g, b).start()

                if j == 0:
                    @pl.when(k >= 1)
                    def _():
                        gather(g - 1, bp).wait()
                        writeback(g - 1, bp).start()
                else:
                    gather(g - 1, bp).wait()
                    writeback(g - 1, bp).start()
            return ()

        lax.fori_loop(0, n_chunks // 2, outer, (), unroll=False)

        gather(n_chunks - 1, 1).wait()
        writeback(n_chunks - 1, 1).start()
        writeback(n_chunks - 2, 0).wait()
        writeback(n_chunks - 1, 1).wait()

    return gather_kernel


@jax.jit
def kernel(inputs, embedding):
    B = inputs.size
    flat = inputs.reshape(B)
    out = _build(B)(flat, embedding)
    return out.reshape(inputs.shape + (embedding.shape[1],))


# speedup vs baseline: 1.2224x; 1.2214x over previous
import functools

import jax
import jax.numpy as jnp
from jax import lax
from jax.experimental import pallas as pl
from jax.experimental.pallas import tpu as pltpu
from jax.experimental.pallas import tpu_sc as plsc

_NC = 2
_NS = 16
_NW = _NC * _NS
_DP = 128
_C = 256


@functools.lru_cache(maxsize=None)
def _build(B, V):
    assert B % (_NW * 2 * _C) == 0
    b_per_w = B // _NW
    n_chunks = b_per_w // _C
    mesh = plsc.VectorSubcoreMesh(
        core_axis_name="c", subcore_axis_name="s",
        num_cores=_NC, num_subcores=_NS)

    @functools.partial(
        pl.kernel,
        out_type=jax.ShapeDtypeStruct((B, _DP), jnp.float32),
        mesh=mesh,
        scratch_types=[
            pltpu.VMEM((b_per_w,), jnp.int32),
            pltpu.VMEM((_C, _DP), jnp.float32),
            pltpu.VMEM((_C, _DP), jnp.float32),
            pltpu.SemaphoreType.DMA,
            pltpu.SemaphoreType.DMA,
            pltpu.SemaphoreType.DMA,
            pltpu.SemaphoreType.DMA,
        ],
        compiler_params=pltpu.CompilerParams(use_tc_tiling_on_sc=False),
    )
    def gather_kernel(idx_hbm, table_hbm, out_hbm,
                      idx_v, rows0, rows1, sg0, sg1, so0, so1):
        wid = lax.axis_index("s") * _NC + lax.axis_index("c")
        base = wid * b_per_w
        bufs = ((rows0, sg0, so0), (rows1, sg1, so1))

        pltpu.sync_copy(idx_hbm.at[pl.ds(pl.multiple_of(base, _C), b_per_w)],
                        idx_v)

        def gather(g, b):
            rows, sg, _ = bufs[b]
            loc = pl.multiple_of(g * _C, _C)
            return pltpu.make_async_copy(
                table_hbm.at[idx_v.at[pl.ds(loc, _C)]], rows, sg)

        def writeback(g, b):
            rows, _, so = bufs[b]
            off = pl.multiple_of(base + g * _C, _C)
            return pltpu.make_async_copy(rows, out_hbm.at[pl.ds(off, _C)], so)

        def outer(k, _):
            for j in range(2):
                g = 2 * k + j
                b = j
                bp = 1 - j

                @pl.when(k >= 1)
                def _():
                    writeback(g - 2, b).wait()
                gather(g, b).start()

                if j == 0:
                    @pl.when(k >= 1)
                    def _():
                        gather(g - 1, bp).wait()
                        writeback(g - 1, bp).start()
                else:
                    gather(g - 1, bp).wait()
                    writeback(g - 1, bp).start()
            return ()

        lax.fori_loop(0, n_chunks // 2, outer, (), unroll=False)

        gather(n_chunks - 1, 1).wait()
        writeback(n_chunks - 1, 1).start()
        writeback(n_chunks - 2, 0).wait()
        writeback(n_chunks - 1, 1).wait()

    return gather_kernel


@jax.jit
def kernel(inputs, embedding):
    B = inputs.size
    V, D = embedding.shape
    flat = inputs.reshape(B)
    table_p = jnp.pad(embedding, ((0, 0), (0, _DP - D)))
    out_p = _build(B, V)(flat, table_p)
    return out_p[:, :D].reshape(inputs.shape + (D,))


# speedup vs baseline: 1.3237x; 1.0829x over previous
import functools

import jax
import jax.numpy as jnp
from jax import lax
from jax.experimental import pallas as pl
from jax.experimental.pallas import tpu as pltpu
from jax.experimental.pallas import tpu_sc as plsc

_NC = 2
_NS = 16
_NW = _NC * _NS
_DP = 128
_C = 320


@functools.lru_cache(maxsize=None)
def _build(B, V):
    assert B % (_NW * 2 * _C) == 0
    b_per_w = B // _NW
    n_chunks = b_per_w // _C
    mesh = plsc.VectorSubcoreMesh(
        core_axis_name="c", subcore_axis_name="s",
        num_cores=_NC, num_subcores=_NS)

    @functools.partial(
        pl.kernel,
        out_type=jax.ShapeDtypeStruct((B, _DP), jnp.float32),
        mesh=mesh,
        scratch_types=[
            pltpu.VMEM((b_per_w,), jnp.int32),
            pltpu.VMEM((_C, _DP), jnp.float32),
            pltpu.VMEM((_C, _DP), jnp.float32),
            pltpu.SemaphoreType.DMA,
            pltpu.SemaphoreType.DMA,
            pltpu.SemaphoreType.DMA,
            pltpu.SemaphoreType.DMA,
        ],
        compiler_params=pltpu.CompilerParams(use_tc_tiling_on_sc=False),
    )
    def gather_kernel(idx_hbm, table_hbm, out_hbm,
                      idx_v, rows0, rows1, sg0, sg1, so0, so1):
        wid = lax.axis_index("s") * _NC + lax.axis_index("c")
        base = wid * b_per_w
        bufs = ((rows0, sg0, so0), (rows1, sg1, so1))

        pltpu.sync_copy(idx_hbm.at[pl.ds(pl.multiple_of(base, _C), b_per_w)],
                        idx_v)

        def gather(g, b):
            rows, sg, _ = bufs[b]
            loc = pl.multiple_of(g * _C, _C)
            return pltpu.make_async_copy(
                table_hbm.at[idx_v.at[pl.ds(loc, _C)]], rows, sg)

        def writeback(g, b):
            rows, _, so = bufs[b]
            off = pl.multiple_of(base + g * _C, _C)
            return pltpu.make_async_copy(
                rows.at[:, pl.ds(0, 64)],
                out_hbm.at[pl.ds(off, _C), pl.ds(0, 64)], so)

        def outer(k, _):
            for j in range(2):
                g = 2 * k + j
                b = j
                bp = 1 - j

                @pl.when(k >= 1)
                def _():
                    writeback(g - 2, b).wait()
                gather(g, b).start()

                if j == 0:
                    @pl.when(k >= 1)
                    def _():
                        gather(g - 1, bp).wait()
                        writeback(g - 1, bp).start()
                else:
                    gather(g - 1, bp).wait()
                    writeback(g - 1, bp).start()
            return ()

        lax.fori_loop(0, n_chunks // 2, outer, (), unroll=False)

        gather(n_chunks - 1, 1).wait()
        writeback(n_chunks - 1, 1).start()
        writeback(n_chunks - 2, 0).wait()
        writeback(n_chunks - 1, 1).wait()

    return gather_kernel


@jax.jit
def kernel(inputs, embedding):
    B = inputs.size
    V, D = embedding.shape
    flat = inputs.reshape(B)
    table_p = jnp.pad(embedding, ((0, 0), (0, _DP - D)))
    out_p = _build(B, V)(flat, table_p)
    return out_p[:, :D].reshape(inputs.shape + (D,))


# speedup vs baseline: 1.4349x; 1.0840x over previous
import functools

import jax
import jax.numpy as jnp
from jax import lax
from jax.experimental import pallas as pl
from jax.experimental.pallas import tpu as pltpu
from jax.experimental.pallas import tpu_sc as plsc

_NC = 2
_NS = 16
_NW = _NC * _NS
_DP = 128
_C = 512


@functools.lru_cache(maxsize=None)
def _build(B, V):
    assert B % (_NW * 2 * _C) == 0
    b_per_w = B // _NW
    n_chunks = b_per_w // _C
    mesh = plsc.VectorSubcoreMesh(
        core_axis_name="c", subcore_axis_name="s",
        num_cores=_NC, num_subcores=_NS)

    @functools.partial(
        pl.kernel,
        out_type=jax.ShapeDtypeStruct((B, _DP), jnp.float32),
        mesh=mesh,
        scratch_types=[
            pltpu.VMEM((b_per_w,), jnp.int32),
            pltpu.VMEM((_C, 64), jnp.float32),
            pltpu.VMEM((_C, 64), jnp.float32),
            pltpu.SemaphoreType.DMA,
            pltpu.SemaphoreType.DMA,
            pltpu.SemaphoreType.DMA,
            pltpu.SemaphoreType.DMA,
        ],
        compiler_params=pltpu.CompilerParams(use_tc_tiling_on_sc=False),
    )
    def gather_kernel(idx_hbm, table_hbm, out_hbm,
                      idx_v, rows0, rows1, sg0, sg1, so0, so1):
        wid = lax.axis_index("s") * _NC + lax.axis_index("c")
        base = wid * b_per_w
        bufs = ((rows0, sg0, so0), (rows1, sg1, so1))

        pltpu.sync_copy(idx_hbm.at[pl.ds(pl.multiple_of(base, _C), b_per_w)],
                        idx_v)

        def dbl(i, _):
            sl = pl.ds(i * 16, 16)
            idx_v[sl] = idx_v[sl] * 2
            return ()

        lax.fori_loop(0, b_per_w // 16, dbl, (), unroll=4)

        def gather(g, b):
            rows, sg, _ = bufs[b]
            loc = pl.multiple_of(g * _C, _C)
            return pltpu.make_async_copy(
                table_hbm.at[idx_v.at[pl.ds(loc, _C)]], rows, sg)

        def writeback(g, b):
            rows, _, so = bufs[b]
            off = pl.multiple_of(base + g * _C, _C)
            return pltpu.make_async_copy(
                rows, out_hbm.at[pl.ds(off, _C), pl.ds(0, 64)], so)

        def outer(k, _):
            for j in range(2):
                g = 2 * k + j
                b = j
                bp = 1 - j

                @pl.when(k >= 1)
                def _():
                    writeback(g - 2, b).wait()
                gather(g, b).start()

                if j == 0:
                    @pl.when(k >= 1)
                    def _():
                        gather(g - 1, bp).wait()
                        writeback(g - 1, bp).start()
                else:
                    gather(g - 1, bp).wait()
                    writeback(g - 1, bp).start()
            return ()

        lax.fori_loop(0, n_chunks // 2, outer, (), unroll=False)

        gather(n_chunks - 1, 1).wait()
        writeback(n_chunks - 1, 1).start()
        writeback(n_chunks - 2, 0).wait()
        writeback(n_chunks - 1, 1).wait()

    return gather_kernel


@jax.jit
def kernel(inputs, embedding):
    B = inputs.size
    V, D = embedding.shape
    flat = inputs.reshape(B)
    table_p = jnp.pad(embedding, ((0, 0), (0, _DP - D)))
    table2 = table_p.reshape(2 * V, D)
    out_p = _build(B, V)(flat, table2)
    return out_p[:, :D].reshape(inputs.shape + (D,))
